# baseline (speedup 1.0000x reference)
import jax
import jax.numpy as jnp
from jax.experimental import pallas as pl
from jax.experimental.pallas import tpu as pltpu

_TM = 256
_CORES = 2


def _mm_bias_kernel(x_ref, w_ref, b_ref, o_ref, wb_ref):
    @pl.when(pl.program_id(1) == 0)
    def _cast_w():
        wb_ref[...] = w_ref[...].astype(jnp.bfloat16)

    xb = x_ref[...].astype(jnp.bfloat16)
    acc = jnp.dot(xb, wb_ref[...], preferred_element_type=jnp.float32)
    o_ref[...] = acc + b_ref[...]


def kernel(x, weight, bias):
    *lead, nx = x.shape
    nf = weight.shape[1]
    x2d = x.reshape(-1, nx)
    m = x2d.shape[0]
    inner = m // _TM // _CORES
    out = pl.pallas_call(
        _mm_bias_kernel,
        out_shape=jax.ShapeDtypeStruct((m, nf), x.dtype),
        grid=(_CORES, inner),
        in_specs=[
            pl.BlockSpec((_TM, nx), lambda c, j: (c * inner + j, 0)),
            pl.BlockSpec((nx, nf), lambda c, j: (0, 0)),
            pl.BlockSpec((1, nf), lambda c, j: (0, 0)),
        ],
        out_specs=pl.BlockSpec((_TM, nf), lambda c, j: (c * inner + j, 0)),
        scratch_shapes=[pltpu.VMEM((nx, nf), jnp.bfloat16)],
        compiler_params=pltpu.CompilerParams(
            dimension_semantics=("parallel", "arbitrary"),
            vmem_limit_bytes=56 << 20,
        ),
    )(x2d, weight, bias.reshape(1, nf))
    return out.reshape(*lead, nf)


# speedup vs baseline: 1.1788x; 1.1788x over previous
import jax
import jax.numpy as jnp
from jax.experimental import pallas as pl
from jax.experimental.pallas import tpu as pltpu

_TM = 512


def _mm_bias_kernel(x_ref, w_ref, b_ref, o_ref):
    xb = x_ref[...].astype(jnp.bfloat16)
    wb = w_ref[...].astype(jnp.bfloat16)
    acc = jnp.dot(xb, wb, preferred_element_type=jnp.float32)
    o_ref[...] = acc + b_ref[...]


def kernel(x, weight, bias):
    *lead, nx = x.shape
    nf = weight.shape[1]
    x2d = x.reshape(-1, nx)
    m = x2d.shape[0]
    out = pl.pallas_call(
        _mm_bias_kernel,
        out_shape=jax.ShapeDtypeStruct((m, nf), x.dtype),
        grid=(m // _TM,),
        in_specs=[
            pl.BlockSpec((_TM, nx), lambda i: (i, 0)),
            pl.BlockSpec((nx, nf), lambda i: (0, 0)),
            pl.BlockSpec((1, nf), lambda i: (0, 0)),
        ],
        out_specs=pl.BlockSpec((_TM, nf), lambda i: (i, 0)),
        compiler_params=pltpu.CompilerParams(
            dimension_semantics=("parallel",),
            vmem_limit_bytes=64 << 20,
        ),
    )(x2d, weight, bias.reshape(1, nf))
    return out.reshape(*lead, nf)
